# Initial kernel scaffold; baseline (speedup 1.0000x reference)
import jax
import jax.numpy as jnp
from jax.experimental import pallas as pl


def kernel(anchor_points_lvl0, anchor_points_lvl1, anchor_points_lvl2, anchor_points_lvl3, anchor_points_lvl4, gt_bboxes, mean, sigma, labels, inside_gt_bbox_mask):
    raise NotImplementedError("write your pallas kernel here")


# speedup vs baseline: 2.4712x; 2.4712x over previous
import functools

import jax
import jax.numpy as jnp
from jax.experimental import pallas as pl

_STRIDES = (8, 16, 32, 64, 128)
_LEVEL_SIZES = (4096, 1024, 256, 64, 16)
_NUM_POINTS = sum(_LEVEL_SIZES)
_NUM_GT = 200
_NUM_CLASSES = 80
_TOPK = 9


def _center_prior_kernel(px_ref, py_ref, inv_s_ref, gtb_t_ref, mean_ref,
                         sigma_ref, labels_ref, inside_ref,
                         out_w_ref, out_m_ref):
    n = _NUM_POINTS
    g = _NUM_GT

    gtb = gtb_t_ref[...]
    cx = (gtb[0:1, :] + gtb[2:3, :]) * 0.5
    cy = (gtb[1:2, :] + gtb[3:4, :]) * 0.5

    lab = labels_ref[...]
    cls_iota = jax.lax.broadcasted_iota(jnp.int32, (_NUM_CLASSES, g), 0)
    oh = cls_iota == lab
    zero_cg = jnp.zeros((_NUM_CLASSES, g), jnp.float32)
    mx = jnp.sum(jnp.where(oh, mean_ref[:, 0:1], zero_cg), axis=0, keepdims=True)
    my = jnp.sum(jnp.where(oh, mean_ref[:, 1:2], zero_cg), axis=0, keepdims=True)
    sx = jnp.sum(jnp.where(oh, sigma_ref[:, 0:1], zero_cg), axis=0, keepdims=True)
    sy = jnp.sum(jnp.where(oh, sigma_ref[:, 1:2], zero_cg), axis=0, keepdims=True)
    den_x = 2.0 * sx * sx
    den_y = 2.0 * sy * sy

    px = px_ref[...]
    py = py_ref[...]
    inv_s = inv_s_ref[...]
    dx = (px - cx) * inv_s - mx
    dy = (py - cy) * inv_s - my
    w = jnp.exp(-(dx * dx) / den_x) * jnp.exp(-(dy * dy) / den_y)

    inside = inside_ref[...]
    cnt = jnp.sum(inside, axis=0, keepdims=True)
    no_pts = cnt == 0.0

    row_iota = jax.lax.broadcasted_iota(jnp.int32, (n, g), 0)
    cur = w
    taken = jnp.zeros((n, g), jnp.float32)
    for _ in range(_TOPK):
        m = jnp.max(cur, axis=0, keepdims=True)
        idx = jnp.min(jnp.where(cur == m, row_iota, n), axis=0, keepdims=True)
        sel = row_iota == idx
        taken = jnp.where(sel, 1.0, taken)
        cur = jnp.where(sel, -1.0, cur)

    inside_mask = jnp.where(no_pts, taken, inside)
    out_m_ref[...] = inside_mask
    out_w_ref[...] = w * inside_mask


@functools.partial(jax.jit, static_argnames=())
def kernel(anchor_points_lvl0, anchor_points_lvl1, anchor_points_lvl2,
           anchor_points_lvl3, anchor_points_lvl4, gt_bboxes, mean, sigma,
           labels, inside_gt_bbox_mask):
    pts = jnp.concatenate([anchor_points_lvl0, anchor_points_lvl1,
                           anchor_points_lvl2, anchor_points_lvl3,
                           anchor_points_lvl4], axis=0)
    px = pts[:, 0:1]
    py = pts[:, 1:2]
    inv_s = jnp.concatenate([
        jnp.full((sz, 1), 1.0 / s, jnp.float32)
        for sz, s in zip(_LEVEL_SIZES, _STRIDES)], axis=0)
    gtb_t = gt_bboxes.T
    labels2 = labels.reshape(1, _NUM_GT).astype(jnp.int32)
    inside_f = inside_gt_bbox_mask.astype(jnp.float32)

    out_w, out_m = pl.pallas_call(
        _center_prior_kernel,
        out_shape=(
            jax.ShapeDtypeStruct((_NUM_POINTS, _NUM_GT), jnp.float32),
            jax.ShapeDtypeStruct((_NUM_POINTS, _NUM_GT), jnp.float32),
        ),
    )(px, py, inv_s, gtb_t, mean, sigma, labels2, inside_f)
    return out_w, out_m.astype(jnp.bool_)


# speedup vs baseline: 2.9190x; 1.1812x over previous
import functools

import jax
import jax.numpy as jnp
from jax.experimental import pallas as pl

_STRIDES = (8, 16, 32, 64, 128)
_LEVEL_SIZES = (4096, 1024, 256, 64, 16)
_NUM_POINTS = sum(_LEVEL_SIZES)
_NUM_GT = 200
_NUM_CLASSES = 80
_TOPK = 9


def _center_prior_kernel(px_ref, py_ref, inv_s_ref, gtb_t_ref, mean_ref,
                         sigma_ref, labels_ref, inside_ref,
                         out_w_ref, out_m_ref):
    n = _NUM_POINTS
    g = _NUM_GT

    gtb = gtb_t_ref[...]
    cx = (gtb[0:1, :] + gtb[2:3, :]) * 0.5
    cy = (gtb[1:2, :] + gtb[3:4, :]) * 0.5

    lab = labels_ref[...]
    cls_iota = jax.lax.broadcasted_iota(jnp.int32, (_NUM_CLASSES, g), 0)
    oh = cls_iota == lab
    zero_cg = jnp.zeros((_NUM_CLASSES, g), jnp.float32)
    mx = jnp.sum(jnp.where(oh, mean_ref[:, 0:1], zero_cg), axis=0, keepdims=True)
    my = jnp.sum(jnp.where(oh, mean_ref[:, 1:2], zero_cg), axis=0, keepdims=True)
    sx = jnp.sum(jnp.where(oh, sigma_ref[:, 0:1], zero_cg), axis=0, keepdims=True)
    sy = jnp.sum(jnp.where(oh, sigma_ref[:, 1:2], zero_cg), axis=0, keepdims=True)
    nax = -1.0 / (2.0 * sx * sx)
    nay = -1.0 / (2.0 * sy * sy)

    px = px_ref[...]
    py = py_ref[...]
    inv_s = inv_s_ref[...]
    dx = (px - cx) * inv_s - mx
    dy = (py - cy) * inv_s - my
    w = jnp.exp(dx * dx * nax + dy * dy * nay)

    inside = inside_ref[...]
    cnt = jnp.sum(inside, axis=0, keepdims=True)
    no_pts = cnt == 0.0

    row_iota = jax.lax.broadcasted_iota(jnp.int32, (n, g), 0)
    cur = w
    for _ in range(_TOPK):
        idx = jnp.argmax(cur, axis=0, keepdims=True)
        sel = row_iota == idx
        cur = jnp.where(sel, -1.0, cur)
    taken = jnp.where(cur < 0.0, 1.0, 0.0)

    inside_mask = jnp.where(no_pts, taken, inside)
    out_m_ref[...] = inside_mask
    out_w_ref[...] = w * inside_mask


@functools.partial(jax.jit, static_argnames=())
def kernel(anchor_points_lvl0, anchor_points_lvl1, anchor_points_lvl2,
           anchor_points_lvl3, anchor_points_lvl4, gt_bboxes, mean, sigma,
           labels, inside_gt_bbox_mask):
    pts = jnp.concatenate([anchor_points_lvl0, anchor_points_lvl1,
                           anchor_points_lvl2, anchor_points_lvl3,
                           anchor_points_lvl4], axis=0)
    px = pts[:, 0:1]
    py = pts[:, 1:2]
    inv_s = jnp.concatenate([
        jnp.full((sz, 1), 1.0 / s, jnp.float32)
        for sz, s in zip(_LEVEL_SIZES, _STRIDES)], axis=0)
    gtb_t = gt_bboxes.T
    labels2 = labels.reshape(1, _NUM_GT).astype(jnp.int32)
    inside_f = inside_gt_bbox_mask.astype(jnp.float32)

    out_w, out_m = pl.pallas_call(
        _center_prior_kernel,
        out_shape=(
            jax.ShapeDtypeStruct((_NUM_POINTS, _NUM_GT), jnp.float32),
            jax.ShapeDtypeStruct((_NUM_POINTS, _NUM_GT), jnp.float32),
        ),
    )(px, py, inv_s, gtb_t, mean, sigma, labels2, inside_f)
    return out_w, out_m.astype(jnp.bool_)


# speedup vs baseline: 2.9762x; 1.0196x over previous
import functools

import jax
import jax.numpy as jnp
from jax.experimental import pallas as pl

_STRIDES = (8, 16, 32, 64, 128)
_LEVEL_SIZES = (4096, 1024, 256, 64, 16)
_NUM_POINTS = sum(_LEVEL_SIZES)
_NUM_GT = 200
_NUM_CLASSES = 80
_TOPK = 9


def _center_prior_kernel(px_ref, py_ref, inv_s_ref, gtb_t_ref, msig_t_ref,
                         labels_ref, inside_ref, out_w_ref, out_m_ref):
    n = _NUM_POINTS
    g = _NUM_GT

    gtb = gtb_t_ref[...]
    cx = (gtb[0:1, :] + gtb[2:3, :]) * 0.5
    cy = (gtb[1:2, :] + gtb[3:4, :]) * 0.5

    lab = labels_ref[...]
    cls_iota = jax.lax.broadcasted_iota(jnp.int32, (_NUM_CLASSES, g), 0)
    oh = (cls_iota == lab).astype(jnp.float32)
    msig = jax.lax.dot_general(
        msig_t_ref[...], oh, (((1,), (0,)), ((), ())),
        precision=jax.lax.Precision.HIGHEST,
        preferred_element_type=jnp.float32)
    mx = msig[0:1, :]
    my = msig[1:2, :]
    sx = msig[2:3, :]
    sy = msig[3:4, :]
    nax = -1.0 / (2.0 * sx * sx)
    nay = -1.0 / (2.0 * sy * sy)

    px = px_ref[...]
    py = py_ref[...]
    inv_s = inv_s_ref[...]
    dx = (px - cx) * inv_s - mx
    dy = (py - cy) * inv_s - my
    w = jnp.exp(dx * dx * nax + dy * dy * nay)

    inside = jnp.where(inside_ref[...], 1.0, 0.0)
    cnt = jnp.sum(inside, axis=0, keepdims=True)
    no_pts = cnt == 0.0

    row_iota = jax.lax.broadcasted_iota(jnp.int32, (n, g), 0)
    cur = w
    for _ in range(_TOPK):
        idx = jnp.argmax(cur, axis=0, keepdims=True)
        sel = row_iota == idx
        cur = jnp.where(sel, -1.0, cur)
    taken = jnp.where(cur < 0.0, 1.0, 0.0)

    inside_mask = jnp.where(no_pts, taken, inside)
    out_m_ref[...] = inside_mask > 0.5
    out_w_ref[...] = w * inside_mask


@functools.partial(jax.jit, static_argnames=())
def kernel(anchor_points_lvl0, anchor_points_lvl1, anchor_points_lvl2,
           anchor_points_lvl3, anchor_points_lvl4, gt_bboxes, mean, sigma,
           labels, inside_gt_bbox_mask):
    pts = jnp.concatenate([anchor_points_lvl0, anchor_points_lvl1,
                           anchor_points_lvl2, anchor_points_lvl3,
                           anchor_points_lvl4], axis=0)
    px = pts[:, 0:1]
    py = pts[:, 1:2]
    inv_s = jnp.concatenate([
        jnp.full((sz, 1), 1.0 / s, jnp.float32)
        for sz, s in zip(_LEVEL_SIZES, _STRIDES)], axis=0)
    gtb_t = gt_bboxes.T
    msig_t = jnp.concatenate([mean.T, sigma.T], axis=0)
    labels2 = labels.reshape(1, _NUM_GT).astype(jnp.int32)

    out_w, out_m = pl.pallas_call(
        _center_prior_kernel,
        out_shape=(
            jax.ShapeDtypeStruct((_NUM_POINTS, _NUM_GT), jnp.float32),
            jax.ShapeDtypeStruct((_NUM_POINTS, _NUM_GT), jnp.bool_),
        ),
    )(px, py, inv_s, gtb_t, msig_t, labels2, inside_gt_bbox_mask)
    return out_w, out_m


# speedup vs baseline: 3.5100x; 1.1794x over previous
import functools

import jax
import jax.numpy as jnp
from jax.experimental import pallas as pl

_STRIDES = (8, 16, 32, 64, 128)
_LEVEL_SIZES = (4096, 1024, 256, 64, 16)
_NUM_POINTS = sum(_LEVEL_SIZES)
_NUM_GT = 200
_NUM_CLASSES = 80
_TOPK = 9
_GBLK = 128


def _center_prior_kernel(pts_ref, gtb_t_ref, msig_t_ref, labels_ref,
                         inside_ref, out_w_ref, out_m_ref):
    n = _NUM_POINTS
    g = _GBLK
    j = pl.program_id(0)

    gtb = gtb_t_ref[...]
    cx = (gtb[0:1, :] + gtb[2:3, :]) * 0.5
    cy = (gtb[1:2, :] + gtb[3:4, :]) * 0.5

    lab = labels_ref[...]
    cls_iota = jax.lax.broadcasted_iota(jnp.int32, (_NUM_CLASSES, g), 0)
    oh = (cls_iota == lab).astype(jnp.float32)
    msig = jax.lax.dot_general(
        msig_t_ref[...], oh, (((1,), (0,)), ((), ())),
        precision=jax.lax.Precision.HIGHEST,
        preferred_element_type=jnp.float32)
    mx = msig[0:1, :]
    my = msig[1:2, :]
    nax = -1.0 / (2.0 * msig[2:3, :] * msig[2:3, :])
    nay = -1.0 / (2.0 * msig[3:4, :] * msig[3:4, :])

    row1 = jax.lax.broadcasted_iota(jnp.int32, (n, 1), 0)
    bound = _LEVEL_SIZES[0]
    inv_s = jnp.full((n, 1), 1.0 / _STRIDES[0], jnp.float32)
    for sz, s in zip(_LEVEL_SIZES[1:], _STRIDES[1:]):
        inv_s = jnp.where(row1 >= bound, 1.0 / s, inv_s)
        bound += sz

    pts = pts_ref[...]
    px = pts[:, 0:1]
    py = pts[:, 1:2]
    dx = (px - cx) * inv_s - mx
    dy = (py - cy) * inv_s - my
    w = jnp.exp(dx * dx * nax + dy * dy * nay)

    inside = jnp.where(inside_ref[...], 1.0, 0.0)
    cnt = jnp.sum(inside, axis=0, keepdims=True)
    col_valid = jax.lax.broadcasted_iota(jnp.int32, (1, g), 1) + j * g < _NUM_GT
    no_pts = jnp.logical_and(cnt == 0.0, col_valid)
    need_topk = jnp.any(no_pts)

    @pl.when(need_topk)
    def _force_topk_path():
        row_iota = jax.lax.broadcasted_iota(jnp.int32, (n, g), 0)
        cur = w
        for _ in range(_TOPK):
            idx = jnp.argmax(cur, axis=0, keepdims=True)
            cur = jnp.where(row_iota == idx, -1.0, cur)
        taken = jnp.where(cur < 0.0, 1.0, 0.0)
        inside_mask = jnp.where(no_pts, taken, inside)
        out_m_ref[...] = inside_mask > 0.5
        out_w_ref[...] = w * inside_mask

    @pl.when(jnp.logical_not(need_topk))
    def _plain_path():
        out_m_ref[...] = inside > 0.5
        out_w_ref[...] = w * inside


@functools.partial(jax.jit, static_argnames=())
def kernel(anchor_points_lvl0, anchor_points_lvl1, anchor_points_lvl2,
           anchor_points_lvl3, anchor_points_lvl4, gt_bboxes, mean, sigma,
           labels, inside_gt_bbox_mask):
    pts = jnp.concatenate([anchor_points_lvl0, anchor_points_lvl1,
                           anchor_points_lvl2, anchor_points_lvl3,
                           anchor_points_lvl4], axis=0)
    gtb_t = gt_bboxes.T
    msig_t = jnp.concatenate([mean.T, sigma.T], axis=0)
    labels2 = labels.reshape(1, _NUM_GT).astype(jnp.int32)

    nblk = (_NUM_GT + _GBLK - 1) // _GBLK
    out_w, out_m = pl.pallas_call(
        _center_prior_kernel,
        grid=(nblk,),
        in_specs=[
            pl.BlockSpec((_NUM_POINTS, 2), lambda j: (0, 0)),
            pl.BlockSpec((4, _GBLK), lambda j: (0, j)),
            pl.BlockSpec((4, _NUM_CLASSES), lambda j: (0, 0)),
            pl.BlockSpec((1, _GBLK), lambda j: (0, j)),
            pl.BlockSpec((_NUM_POINTS, _GBLK), lambda j: (0, j)),
        ],
        out_specs=(
            pl.BlockSpec((_NUM_POINTS, _GBLK), lambda j: (0, j)),
            pl.BlockSpec((_NUM_POINTS, _GBLK), lambda j: (0, j)),
        ),
        out_shape=(
            jax.ShapeDtypeStruct((_NUM_POINTS, _NUM_GT), jnp.float32),
            jax.ShapeDtypeStruct((_NUM_POINTS, _NUM_GT), jnp.bool_),
        ),
    )(pts, gtb_t, msig_t, labels2, inside_gt_bbox_mask)
    return out_w, out_m
